# Initial kernel scaffold; baseline (speedup 1.0000x reference)
import jax
import jax.numpy as jnp
from jax.experimental import pallas as pl


def kernel(feat_index, feat_value, first_weights, feat_embeddings, bias):
    raise NotImplementedError("write your pallas kernel here")


# speedup vs baseline: 1.1808x; 1.1808x over previous
import functools

import jax
import jax.numpy as jnp
from jax import lax
from jax.experimental import pallas as pl
from jax.experimental.pallas import tpu as pltpu
from jax.experimental.pallas import tpu_sc as plsc

NUM_FEAT = 1000000
NUM_FIELD = 26
EMBED = 16
BATCH = 16384

NC = 2
NS = 16
L = 16
NW = NC * NS
ROWS_W = BATCH // NW
CHUNK = 128
NCHUNK = ROWS_W // CHUNK
GPC = CHUNK // L
IDX_PER_CHUNK = CHUNK * NUM_FIELD
NJ = IDX_PER_CHUNK // 128


def _body(idx_hbm, fv_hbm, fw_hbm, emb_hbm, bias_hbm, out_hbm,
          idx_v, rows_v, fwr_v, fv_v, out_v, bias_v, sem):
    cid = lax.axis_index("c")
    sid = lax.axis_index("s")
    wid = sid * NC + cid

    pltpu.sync_copy(fv_hbm.at[wid], fv_v)
    pltpu.sync_copy(bias_hbm, bias_v)
    bias_vec = bias_v[...]
    lanes = lax.iota(jnp.int32, L)
    zero = jnp.zeros((L,), jnp.float32)

    for c_i in range(NCHUNK):
        pltpu.sync_copy(idx_hbm.at[wid, c_i], idx_v)
        copies = []
        for j in range(NJ):
            copies.append(pltpu.async_copy(emb_hbm.at[idx_v.at[j]], rows_v.at[j], sem))
            copies.append(pltpu.async_copy(fw_hbm.at[idx_v.at[j]],
                                           fwr_v.at[pl.ds(j * 128, 128)], sem))
        for cp in copies:
            cp.wait()

        def group_body(g, _, c_i=c_i):
            base_l = g * L

            def f_body(f, carry):
                y1, ssq = carry[0], carry[1]
                ss = carry[2:]
                p = (base_l + lanes) * NUM_FIELD + f
                jj = lax.shift_right_logical(p, 7)
                kk = lax.bitwise_and(p, 127)
                off = pl.multiple_of(f * ROWS_W + c_i * CHUNK + base_l, L)
                vf = fv_v[pl.ds(off, L)]
                fwv = plsc.load_gather(fwr_v, [p])
                y1 = y1 + fwv * vf
                new_ss = []
                for e in range(EMBED):
                    ee = jnp.full((L,), e, jnp.int32)
                    r = plsc.load_gather(rows_v, [jj, kk, ee])
                    t = r * vf
                    new_ss.append(ss[e] + t)
                    ssq = ssq + t * t
                return (y1, ssq) + tuple(new_ss)

            init = (zero, zero) + tuple(zero for _ in range(EMBED))
            res = lax.fori_loop(0, NUM_FIELD, f_body, init)
            y1, ssq = res[0], res[1]
            acc = zero
            for e in range(EMBED):
                acc = acc + res[2 + e] * res[2 + e]
            y2 = 0.5 * (acc - ssq)
            out_v[pl.ds(base_l, L)] = bias_vec + y1 + y2
            return 0

        lax.fori_loop(0, GPC, group_body, 0)
        pltpu.sync_copy(out_v,
                        out_hbm.at[pl.ds(wid * ROWS_W + c_i * CHUNK, CHUNK)])


@functools.partial(jax.jit, static_argnums=())
def _run(idx_r, fv_r, fw_r, emb, bias_r):
    mesh = plsc.VectorSubcoreMesh(core_axis_name="c", subcore_axis_name="s")
    fn = pl.kernel(
        _body,
        out_type=jax.ShapeDtypeStruct((BATCH,), jnp.float32),
        mesh=mesh,
        compiler_params=pltpu.CompilerParams(
            needs_layout_passes=False, use_tc_tiling_on_sc=False),
        scratch_types=[
            pltpu.VMEM((NJ, 128), jnp.int32),
            pltpu.VMEM((NJ, 128, EMBED), jnp.float32),
            pltpu.VMEM((IDX_PER_CHUNK,), jnp.float32),
            pltpu.VMEM((NUM_FIELD * ROWS_W,), jnp.float32),
            pltpu.VMEM((CHUNK,), jnp.float32),
            pltpu.VMEM((L,), jnp.float32),
            pltpu.SemaphoreType.DMA,
        ],
    )
    return fn(idx_r, fv_r, fw_r, emb, bias_r)


def kernel(feat_index, feat_value, first_weights, feat_embeddings, bias):
    idx_r = feat_index.reshape(NW, NCHUNK, NJ, 128)
    fv_r = (feat_value.T.reshape(NUM_FIELD, NW, ROWS_W)
            .transpose(1, 0, 2).reshape(NW, NUM_FIELD * ROWS_W))
    fw_r = first_weights.reshape(-1)
    bias_r = jnp.broadcast_to(bias, (L,))
    out = _run(idx_r, fv_r, fw_r, feat_embeddings, bias_r)
    return out[:, None]


# speedup vs baseline: 1.2971x; 1.0985x over previous
import functools

import jax
import jax.numpy as jnp
from jax import lax
from jax.experimental import pallas as pl
from jax.experimental.pallas import tpu as pltpu
from jax.experimental.pallas import tpu_sc as plsc

NUM_FEAT = 1000000
NUM_FIELD = 26
EMBED = 16
BATCH = 16384

NC = 2
NS = 16
L = 16
NW = NC * NS

ROWS_W = BATCH // NW
CHUNK = 64
NCHUNK = ROWS_W // CHUNK
GPC = CHUNK // L
IDX_PER_CHUNK = CHUNK * NUM_FIELD
NJ = IDX_PER_CHUNK // 128

TCOLS = 1024
FULL_CHUNKS = NUM_FEAT // TCOLS
KTOT = 32
NROUND = KTOT // 2
TAIL_START = FULL_CHUNKS * TCOLS
TAIL_ROWS = NUM_FEAT - TAIL_START
TOUT = TCOLS * EMBED


def _tbody(embT_hbm, tail_hbm, out_hbm, slab_v, out_v, tail_v,
           sem_in0, sem_in1, sem_out0, sem_out1):
    sem_in = (sem_in0, sem_in1)
    sem_out = (sem_out0, sem_out1)
    cid = lax.axis_index("c")
    sid = lax.axis_index("s")
    wid = sid * NC + cid
    lanes = lax.iota(jnp.int32, L)
    h_idx = lax.shift_right_logical(lanes, 3)
    e_idx = lax.bitwise_and(lanes, 7)

    def chunk_of(k):
        return jnp.minimum(wid + NW * k, FULL_CHUNKS - 1)

    def in_descs(k, buf, make):
        col0 = pl.multiple_of(chunk_of(k) * TCOLS, TCOLS)
        ds = []
        for h in range(2):
            for t in range(TCOLS // 128):
                src = embT_hbm.at[pl.ds(h * 8, 8), pl.ds(col0 + t * 128, 128)]
                ds.append(make(src, slab_v.at[buf, h, t], sem_in[buf]))
        return ds

    def out_desc(k, buf, make):
        dst = out_hbm.at[pl.ds(pl.multiple_of(chunk_of(k) * TOUT, TOUT), TOUT)]
        return make(out_v.at[pl.ds(buf * TOUT, TOUT)], dst, sem_out[buf])

    def process(k, buf):
        for d in in_descs(k, buf, pltpu.make_async_copy):
            d.wait()
        p_vec = jnp.full((L,), buf, jnp.int32)
        obase = buf * TOUT

        def jbody(j16, _):
            t_s = lax.shift_right_logical(j16, 3)
            c_base = lax.bitwise_and(j16 * L, 127)
            t_vec = jnp.full((L,), t_s, jnp.int32)
            for u in range(L):
                c_vec = jnp.full((L,), c_base + u, jnp.int32)
                row = plsc.load_gather(
                    slab_v, [p_vec, h_idx, t_vec, e_idx, c_vec])
                out_v[pl.ds(pl.multiple_of(
                    obase + j16 * (L * EMBED) + u * EMBED, L), L)] = row
            return 0

        lax.fori_loop(0, TCOLS // L, jbody, 0)
        out_desc(k, buf, pltpu.async_copy)

    in_descs(0, 0, pltpu.async_copy)
    in_descs(1, 1, pltpu.async_copy)

    def round_body(r, _):
        for b in range(2):
            k = 2 * r + b

            @pl.when(r > 0)
            def _(k=k, b=b):
                out_desc(k - 2, b, pltpu.make_async_copy).wait()

            process(k, b)
            in_descs(k + 2, b, pltpu.async_copy)
        return 0

    lax.fori_loop(0, NROUND, round_body, 0)
    for b in range(2):
        out_desc(KTOT - 2 + b, b, pltpu.make_async_copy).wait()
        for d in in_descs(KTOT + b, b, pltpu.make_async_copy):
            d.wait()

    @pl.when(wid == NW - 1)
    def _():
        pltpu.sync_copy(tail_hbm, tail_v)
        pltpu.sync_copy(tail_v,
                        out_hbm.at[pl.ds(TAIL_START * EMBED,
                                         TAIL_ROWS * EMBED)])


def _gbody(idx_hbm, fv_hbm, fw_hbm, emb_hbm, bias_hbm, out_hbm,
           idx_v, rows_v, fwr_v, fv_v, out_v, bias_v, sem0, sem1):
    sem = (sem0, sem1)
    cid = lax.axis_index("c")
    sid = lax.axis_index("s")
    wid = sid * NC + cid

    pltpu.sync_copy(fv_hbm.at[wid], fv_v)
    pltpu.sync_copy(bias_hbm, bias_v)
    bias_vec = bias_v[...]
    lanes = lax.iota(jnp.int32, L)
    zero = jnp.zeros((L,), jnp.float32)

    def fire_chunk(c_i, buf):
        pltpu.sync_copy(idx_hbm.at[wid, c_i], idx_v.at[buf])
        cps = []
        for j in range(NJ):
            cps.append(pltpu.async_copy(emb_hbm.at[idx_v.at[buf, j]],
                                        rows_v.at[buf, j], sem[buf]))
            cps.append(pltpu.async_copy(fw_hbm.at[idx_v.at[buf, j]],
                                        fwr_v.at[buf, j], sem[buf]))
        return cps

    in_flight = [None, None]
    in_flight[0] = fire_chunk(0, 0)

    for c_i in range(NCHUNK):
        p = c_i % 2
        if c_i + 1 < NCHUNK:
            in_flight[1 - p] = fire_chunk(c_i + 1, 1 - p)
        for cp in in_flight[p]:
            cp.wait()
        p_vec = jnp.full((L,), p, jnp.int32)

        def group_body(g, _, c_i=c_i, p_vec=p_vec):
            base_l = g * L

            def f_body(f, carry):
                y1, ssq = carry[0], carry[1]
                ss = carry[2:]
                p_pos = (base_l + lanes) * NUM_FIELD + f
                jj = lax.shift_right_logical(p_pos, 7)
                kk = lax.bitwise_and(p_pos, 127)
                off = pl.multiple_of(f * ROWS_W + c_i * CHUNK + base_l, L)
                vf = fv_v[pl.ds(off, L)]
                fwv = plsc.load_gather(fwr_v, [p_vec, jj, kk])
                y1 = y1 + fwv * vf
                new_ss = []
                for e in range(EMBED):
                    ee = jnp.full((L,), e, jnp.int32)
                    r = plsc.load_gather(rows_v, [p_vec, jj, kk, ee])
                    t = r * vf
                    new_ss.append(ss[e] + t)
                    ssq = ssq + t * t
                return (y1, ssq) + tuple(new_ss)

            init = (zero, zero) + tuple(zero for _ in range(EMBED))
            res = lax.fori_loop(0, NUM_FIELD, f_body, init)
            y1, ssq = res[0], res[1]
            acc = zero
            for e in range(EMBED):
                acc = acc + res[2 + e] * res[2 + e]
            y2 = 0.5 * (acc - ssq)
            out_v[pl.ds(c_i * CHUNK + base_l, L)] = bias_vec + y1 + y2
            return 0

        lax.fori_loop(0, GPC, group_body, 0)

    pltpu.sync_copy(out_v, out_hbm.at[pl.ds(wid * ROWS_W, ROWS_W)])


@jax.jit
def _run(idx_r, fv_r, fw_r, embT, tail, bias_r):
    mesh = plsc.VectorSubcoreMesh(core_axis_name="c", subcore_axis_name="s")
    tfn = pl.kernel(
        _tbody,
        out_type=jax.ShapeDtypeStruct((NUM_FEAT * EMBED,), jnp.float32),
        mesh=mesh,
        compiler_params=pltpu.CompilerParams(
            needs_layout_passes=False, use_tc_tiling_on_sc=True),
        scratch_types=[
            pltpu.VMEM((2, 2, TCOLS // 128, 8, 128), jnp.float32),
            pltpu.VMEM((2 * TCOLS * EMBED,), jnp.float32),
            pltpu.VMEM((TAIL_ROWS * EMBED,), jnp.float32),
            pltpu.SemaphoreType.DMA,
            pltpu.SemaphoreType.DMA,
            pltpu.SemaphoreType.DMA,
            pltpu.SemaphoreType.DMA,
        ],
    )
    tab_lin = tfn(embT, tail)
    tab = tab_lin.reshape(NUM_FEAT, EMBED)

    gfn = pl.kernel(
        _gbody,
        out_type=jax.ShapeDtypeStruct((BATCH,), jnp.float32),
        mesh=mesh,
        compiler_params=pltpu.CompilerParams(
            needs_layout_passes=False, use_tc_tiling_on_sc=False),
        scratch_types=[
            pltpu.VMEM((2, NJ, 128), jnp.int32),
            pltpu.VMEM((2, NJ, 128, EMBED), jnp.float32),
            pltpu.VMEM((2, NJ, 128), jnp.float32),
            pltpu.VMEM((NUM_FIELD * ROWS_W,), jnp.float32),
            pltpu.VMEM((ROWS_W,), jnp.float32),
            pltpu.VMEM((L,), jnp.float32),
            pltpu.SemaphoreType.DMA,
            pltpu.SemaphoreType.DMA,
        ],
    )
    return gfn(idx_r, fv_r, fw_r, tab, bias_r)


def kernel(feat_index, feat_value, first_weights, feat_embeddings, bias):
    idx_r = feat_index.reshape(NW, NCHUNK, NJ, 128)
    fv_r = (feat_value.T.reshape(NUM_FIELD, NW, ROWS_W)
            .transpose(1, 0, 2).reshape(NW, NUM_FIELD * ROWS_W))
    fw_r = first_weights.reshape(-1)
    embT = feat_embeddings.T
    tail = feat_embeddings[TAIL_START:].reshape(-1)
    bias_r = jnp.broadcast_to(bias, (L,))
    out = _run(idx_r, fv_r, fw_r, embT, tail, bias_r)
    return out[:, None]


# speedup vs baseline: 1.9149x; 1.4763x over previous
import functools

import jax
import jax.numpy as jnp
from jax import lax
from jax.experimental import pallas as pl
from jax.experimental.pallas import tpu as pltpu
from jax.experimental.pallas import tpu_sc as plsc

NUM_FEAT = 1000000
NUM_FIELD = 26
EMBED = 16
BATCH = 16384

NC = 2
NS = 16
L = 16
NW = NC * NS

ROWS_W = BATCH // NW
CHUNK = 64
NCHUNK = ROWS_W // CHUNK
GPC = CHUNK // L
IDX_PER_CHUNK = CHUNK * NUM_FIELD
NJ = IDX_PER_CHUNK // 128

TCOLS = 1024
FULL_CHUNKS = NUM_FEAT // TCOLS
KTOT = 32
NROUND = KTOT // 2
TAIL_START = FULL_CHUNKS * TCOLS
TAIL_ROWS = NUM_FEAT - TAIL_START
TOUT = TCOLS * EMBED


def _tbody(embT_hbm, tail_hbm, out_hbm, slab_v, out_v, tail_v,
           sem_in0, sem_in1, sem_out0, sem_out1):
    sem_in = (sem_in0, sem_in1)
    sem_out = (sem_out0, sem_out1)
    cid = lax.axis_index("c")
    sid = lax.axis_index("s")
    wid = sid * NC + cid
    lanes = lax.iota(jnp.int32, L)
    h_idx = lax.shift_right_logical(lanes, 3)
    e_idx = lax.bitwise_and(lanes, 7)

    def chunk_of(k):
        return jnp.minimum(wid + NW * k, FULL_CHUNKS - 1)

    def in_descs(k, buf, make):
        col0 = pl.multiple_of(chunk_of(k) * TCOLS, TCOLS)
        ds = []
        for h in range(2):
            for t in range(TCOLS // 128):
                src = embT_hbm.at[pl.ds(h * 8, 8), pl.ds(col0 + t * 128, 128)]
                ds.append(make(src, slab_v.at[buf, h, t], sem_in[buf]))
        return ds

    def out_desc(k, buf, make):
        dst = out_hbm.at[pl.ds(pl.multiple_of(chunk_of(k) * TOUT, TOUT), TOUT)]
        return make(out_v.at[pl.ds(buf * TOUT, TOUT)], dst, sem_out[buf])

    def process(k, buf):
        for d in in_descs(k, buf, pltpu.make_async_copy):
            d.wait()
        p_vec = jnp.full((L,), buf, jnp.int32)
        obase = buf * TOUT

        def jbody(j16, _):
            t_s = lax.shift_right_logical(j16, 3)
            c_base = lax.bitwise_and(j16 * L, 127)
            t_vec = jnp.full((L,), t_s, jnp.int32)
            rows = []
            for u in range(L):
                c_vec = jnp.full((L,), c_base + u, jnp.int32)
                rows.append(plsc.load_gather(
                    slab_v, [p_vec, h_idx, t_vec, e_idx, c_vec]))
            for u in range(L):
                out_v[pl.ds(pl.multiple_of(
                    obase + j16 * (L * EMBED) + u * EMBED, L), L)] = rows[u]
            return 0

        lax.fori_loop(0, TCOLS // L, jbody, 0)
        out_desc(k, buf, pltpu.async_copy)

    in_descs(0, 0, pltpu.async_copy)
    in_descs(1, 1, pltpu.async_copy)

    def round_body(r, _):
        for b in range(2):
            k = 2 * r + b

            @pl.when(r > 0)
            def _(k=k, b=b):
                out_desc(k - 2, b, pltpu.make_async_copy).wait()

            process(k, b)
            in_descs(k + 2, b, pltpu.async_copy)
        return 0

    lax.fori_loop(0, NROUND, round_body, 0)
    for b in range(2):
        out_desc(KTOT - 2 + b, b, pltpu.make_async_copy).wait()
        for d in in_descs(KTOT + b, b, pltpu.make_async_copy):
            d.wait()

    @pl.when(wid == NW - 1)
    def _():
        pltpu.sync_copy(tail_hbm, tail_v)
        pltpu.sync_copy(tail_v,
                        out_hbm.at[pl.ds(TAIL_START * EMBED,
                                         TAIL_ROWS * EMBED)])


def _gbody(idx_hbm, fv_hbm, fw_hbm, emb_hbm, bias_hbm, out_hbm,
           idx_v, rows_v, fwr_v, fv_v, out_v, bias_v, sem0, sem1):
    sem = (sem0, sem1)
    cid = lax.axis_index("c")
    sid = lax.axis_index("s")
    wid = sid * NC + cid

    pltpu.sync_copy(fv_hbm.at[wid], fv_v)
    pltpu.sync_copy(bias_hbm, bias_v)
    bias_vec = bias_v[...]
    lanes = lax.iota(jnp.int32, L)
    zero = jnp.zeros((L,), jnp.float32)

    def fire_chunk(c_i, buf):
        pltpu.sync_copy(idx_hbm.at[wid, c_i], idx_v.at[buf])
        cps = []
        for j in range(NJ):
            cps.append(pltpu.async_copy(emb_hbm.at[idx_v.at[buf, j]],
                                        rows_v.at[buf, j], sem[buf]))
            cps.append(pltpu.async_copy(fw_hbm.at[idx_v.at[buf, j]],
                                        fwr_v.at[buf, j], sem[buf]))
        return cps

    in_flight = [None, None]
    in_flight[0] = fire_chunk(0, 0)

    for c_i in range(NCHUNK):
        p = c_i % 2
        if c_i + 1 < NCHUNK:
            in_flight[1 - p] = fire_chunk(c_i + 1, 1 - p)
        for cp in in_flight[p]:
            cp.wait()
        p_vec = jnp.full((L,), p, jnp.int32)

        def group_body(g, _, c_i=c_i, p_vec=p_vec):
            base_l = g * L

            def f_body(f, carry):
                y1, ssq = carry[0], carry[1]
                ss = carry[2:]
                p_pos = (base_l + lanes) * NUM_FIELD + f
                jj = lax.shift_right_logical(p_pos, 7)
                kk = lax.bitwise_and(p_pos, 127)
                off = pl.multiple_of(f * ROWS_W + c_i * CHUNK + base_l, L)
                vf = fv_v[pl.ds(off, L)]
                fwv = plsc.load_gather(fwr_v, [p_vec, jj, kk])
                rs = []
                for e in range(EMBED):
                    ee = jnp.full((L,), e, jnp.int32)
                    rs.append(plsc.load_gather(rows_v, [p_vec, jj, kk, ee]))
                y1 = y1 + fwv * vf
                new_ss = []
                for e in range(EMBED):
                    t = rs[e] * vf
                    new_ss.append(ss[e] + t)
                    ssq = ssq + t * t
                return (y1, ssq) + tuple(new_ss)

            init = (zero, zero) + tuple(zero for _ in range(EMBED))
            res = lax.fori_loop(0, NUM_FIELD, f_body, init)
            y1, ssq = res[0], res[1]
            acc = zero
            for e in range(EMBED):
                acc = acc + res[2 + e] * res[2 + e]
            y2 = 0.5 * (acc - ssq)
            out_v[pl.ds(c_i * CHUNK + base_l, L)] = bias_vec + y1 + y2
            return 0

        lax.fori_loop(0, GPC, group_body, 0)

    pltpu.sync_copy(out_v, out_hbm.at[pl.ds(wid * ROWS_W, ROWS_W)])


@jax.jit
def _run(idx_r, fv_r, fw_r, embT, tail, bias_r):
    mesh = plsc.VectorSubcoreMesh(core_axis_name="c", subcore_axis_name="s")
    tfn = pl.kernel(
        _tbody,
        out_type=jax.ShapeDtypeStruct((NUM_FEAT * EMBED,), jnp.float32),
        mesh=mesh,
        compiler_params=pltpu.CompilerParams(
            needs_layout_passes=False, use_tc_tiling_on_sc=True),
        scratch_types=[
            pltpu.VMEM((2, 2, TCOLS // 128, 8, 128), jnp.float32),
            pltpu.VMEM((2 * TCOLS * EMBED,), jnp.float32),
            pltpu.VMEM((TAIL_ROWS * EMBED,), jnp.float32),
            pltpu.SemaphoreType.DMA,
            pltpu.SemaphoreType.DMA,
            pltpu.SemaphoreType.DMA,
            pltpu.SemaphoreType.DMA,
        ],
    )
    tab_lin = tfn(embT, tail)
    tab = tab_lin.reshape(NUM_FEAT, EMBED)

    gfn = pl.kernel(
        _gbody,
        out_type=jax.ShapeDtypeStruct((BATCH,), jnp.float32),
        mesh=mesh,
        compiler_params=pltpu.CompilerParams(
            needs_layout_passes=False, use_tc_tiling_on_sc=False),
        scratch_types=[
            pltpu.VMEM((2, NJ, 128), jnp.int32),
            pltpu.VMEM((2, NJ, 128, EMBED), jnp.float32),
            pltpu.VMEM((2, NJ, 128), jnp.float32),
            pltpu.VMEM((NUM_FIELD * ROWS_W,), jnp.float32),
            pltpu.VMEM((ROWS_W,), jnp.float32),
            pltpu.VMEM((L,), jnp.float32),
            pltpu.SemaphoreType.DMA,
            pltpu.SemaphoreType.DMA,
        ],
    )
    return gfn(idx_r, fv_r, fw_r, tab, bias_r)


def kernel(feat_index, feat_value, first_weights, feat_embeddings, bias):
    idx_r = feat_index.reshape(NW, NCHUNK, NJ, 128)
    fv_r = (feat_value.T.reshape(NUM_FIELD, NW, ROWS_W)
            .transpose(1, 0, 2).reshape(NW, NUM_FIELD * ROWS_W))
    fw_r = first_weights.reshape(-1)
    embT = feat_embeddings.T
    tail = feat_embeddings[TAIL_START:].reshape(-1)
    bias_r = jnp.broadcast_to(bias, (L,))
    out = _run(idx_r, fv_r, fw_r, embT, tail, bias_r)
    return out[:, None]


# speedup vs baseline: 1.9274x; 1.0065x over previous
import functools

import jax
import jax.numpy as jnp
from jax import lax
from jax.experimental import pallas as pl
from jax.experimental.pallas import tpu as pltpu
from jax.experimental.pallas import tpu_sc as plsc

NUM_FEAT = 1000000
NUM_FIELD = 26
EMBED = 16
BATCH = 16384

NC = 2
NS = 16
L = 16
NW = NC * NS

ROWS_W = BATCH // NW
CHUNK = 64
NCHUNK = ROWS_W // CHUNK
GPC = CHUNK // L
IDX_PER_CHUNK = CHUNK * NUM_FIELD
NJ = IDX_PER_CHUNK // 128

TCOLS = 1024
FULL_CHUNKS = NUM_FEAT // TCOLS
KTOT = 32
NROUND = KTOT // 2
TAIL_START = FULL_CHUNKS * TCOLS
TAIL_ROWS = NUM_FEAT - TAIL_START
TOUT = TCOLS * EMBED


def _tbody(embT_hbm, tail_hbm, out_hbm, slab_v, out_v, tail_v,
           sem_in0, sem_in1, sem_out0, sem_out1):
    sem_in = (sem_in0, sem_in1)
    sem_out = (sem_out0, sem_out1)
    cid = lax.axis_index("c")
    sid = lax.axis_index("s")
    wid = sid * NC + cid
    lanes = lax.iota(jnp.int32, L)
    h_idx = lax.shift_right_logical(lanes, 3)
    e_idx = lax.bitwise_and(lanes, 7)

    def chunk_of(k):
        return jnp.minimum(wid + NW * k, FULL_CHUNKS - 1)

    def in_descs(k, buf, make):
        col0 = pl.multiple_of(chunk_of(k) * TCOLS, TCOLS)
        ds = []
        for h in range(2):
            for t in range(TCOLS // 128):
                src = embT_hbm.at[pl.ds(h * 8, 8), pl.ds(col0 + t * 128, 128)]
                ds.append(make(src, slab_v.at[buf, h, t], sem_in[buf]))
        return ds

    def out_desc(k, buf, make):
        dst = out_hbm.at[pl.ds(pl.multiple_of(chunk_of(k) * TOUT, TOUT), TOUT)]
        return make(out_v.at[pl.ds(buf * TOUT, TOUT)], dst, sem_out[buf])

    def process(k, buf):
        for d in in_descs(k, buf, pltpu.make_async_copy):
            d.wait()
        p_vec = jnp.full((L,), buf, jnp.int32)
        obase = buf * TOUT

        def jbody(j16, _):
            t_s = lax.shift_right_logical(j16, 3)
            c_base = lax.bitwise_and(j16 * L, 127)
            t_vec = jnp.full((L,), t_s, jnp.int32)
            rows = []
            for u in range(L):
                c_vec = jnp.full((L,), c_base + u, jnp.int32)
                rows.append(plsc.load_gather(
                    slab_v, [p_vec, h_idx, t_vec, e_idx, c_vec]))
            for u in range(L):
                out_v[pl.ds(pl.multiple_of(
                    obase + j16 * (L * EMBED) + u * EMBED, L), L)] = rows[u]
            return 0

        lax.fori_loop(0, TCOLS // L, jbody, 0)
        out_desc(k, buf, pltpu.async_copy)

    in_descs(0, 0, pltpu.async_copy)
    in_descs(1, 1, pltpu.async_copy)

    def round_body(r, _):
        for b in range(2):
            k = 2 * r + b

            @pl.when(r > 0)
            def _(k=k, b=b):
                out_desc(k - 2, b, pltpu.make_async_copy).wait()

            process(k, b)
            in_descs(k + 2, b, pltpu.async_copy)
        return 0

    lax.fori_loop(0, NROUND, round_body, 0)
    for b in range(2):
        out_desc(KTOT - 2 + b, b, pltpu.make_async_copy).wait()
        for d in in_descs(KTOT + b, b, pltpu.make_async_copy):
            d.wait()

    @pl.when(wid == NW - 1)
    def _():
        pltpu.sync_copy(tail_hbm, tail_v)
        pltpu.sync_copy(tail_v,
                        out_hbm.at[pl.ds(TAIL_START * EMBED,
                                         TAIL_ROWS * EMBED)])


def _gbody(idx_hbm, fv_hbm, fw_hbm, emb_hbm, bias_hbm, out_hbm,
           idx_v, rows_v, fwr_v, fv_v, out_v, bias_v, sem0, sem1):
    sem = (sem0, sem1)
    cid = lax.axis_index("c")
    sid = lax.axis_index("s")
    wid = sid * NC + cid

    pltpu.sync_copy(fv_hbm.at[wid], fv_v)
    pltpu.sync_copy(bias_hbm, bias_v)
    bias_vec = bias_v[...]
    lanes = lax.iota(jnp.int32, L)
    zero = jnp.zeros((L,), jnp.float32)

    def fire_chunk(c_i, buf):
        pltpu.sync_copy(idx_hbm.at[wid, c_i], idx_v.at[buf])
        return [
            pltpu.async_copy(emb_hbm.at[idx_v.at[buf]],
                             rows_v.at[buf], sem[buf]),
            pltpu.async_copy(fw_hbm.at[idx_v.at[buf]],
                             fwr_v.at[buf], sem[buf]),
        ]

    in_flight = [None, None]
    in_flight[0] = fire_chunk(0, 0)

    for c_i in range(NCHUNK):
        p = c_i % 2
        if c_i + 1 < NCHUNK:
            in_flight[1 - p] = fire_chunk(c_i + 1, 1 - p)
        for cp in in_flight[p]:
            cp.wait()
        p_vec = jnp.full((L,), p, jnp.int32)

        def group_body(g, _, c_i=c_i, p_vec=p_vec):
            base_l = g * L

            def f_body(f, carry):
                y1, ssq = carry[0], carry[1]
                ss = carry[2:]
                p_pos = (base_l + lanes) * NUM_FIELD + f
                off = pl.multiple_of(f * ROWS_W + c_i * CHUNK + base_l, L)
                vf = fv_v[pl.ds(off, L)]
                fwv = plsc.load_gather(fwr_v, [p_vec, p_pos])
                rs = []
                for e in range(EMBED):
                    ee = jnp.full((L,), e, jnp.int32)
                    rs.append(plsc.load_gather(rows_v, [p_vec, p_pos, ee]))
                y1 = y1 + fwv * vf
                new_ss = []
                for e in range(EMBED):
                    t = rs[e] * vf
                    new_ss.append(ss[e] + t)
                    ssq = ssq + t * t
                return (y1, ssq) + tuple(new_ss)

            init = (zero, zero) + tuple(zero for _ in range(EMBED))
            res = lax.fori_loop(0, NUM_FIELD, f_body, init)
            y1, ssq = res[0], res[1]
            acc = zero
            for e in range(EMBED):
                acc = acc + res[2 + e] * res[2 + e]
            y2 = 0.5 * (acc - ssq)
            out_v[pl.ds(c_i * CHUNK + base_l, L)] = bias_vec + y1 + y2
            return 0

        lax.fori_loop(0, GPC, group_body, 0)

    pltpu.sync_copy(out_v, out_hbm.at[pl.ds(wid * ROWS_W, ROWS_W)])


@jax.jit
def _run(idx_r, fv_r, fw_r, embT, tail, bias_r):
    mesh = plsc.VectorSubcoreMesh(core_axis_name="c", subcore_axis_name="s")
    tfn = pl.kernel(
        _tbody,
        out_type=jax.ShapeDtypeStruct((NUM_FEAT * EMBED,), jnp.float32),
        mesh=mesh,
        compiler_params=pltpu.CompilerParams(
            needs_layout_passes=False, use_tc_tiling_on_sc=True),
        scratch_types=[
            pltpu.VMEM((2, 2, TCOLS // 128, 8, 128), jnp.float32),
            pltpu.VMEM((2 * TCOLS * EMBED,), jnp.float32),
            pltpu.VMEM((TAIL_ROWS * EMBED,), jnp.float32),
            pltpu.SemaphoreType.DMA,
            pltpu.SemaphoreType.DMA,
            pltpu.SemaphoreType.DMA,
            pltpu.SemaphoreType.DMA,
        ],
    )
    tab_lin = tfn(embT, tail)
    tab = tab_lin.reshape(NUM_FEAT, EMBED)

    gfn = pl.kernel(
        _gbody,
        out_type=jax.ShapeDtypeStruct((BATCH,), jnp.float32),
        mesh=mesh,
        compiler_params=pltpu.CompilerParams(
            needs_layout_passes=False, use_tc_tiling_on_sc=False),
        scratch_types=[
            pltpu.VMEM((2, IDX_PER_CHUNK), jnp.int32),
            pltpu.VMEM((2, IDX_PER_CHUNK, EMBED), jnp.float32),
            pltpu.VMEM((2, IDX_PER_CHUNK), jnp.float32),
            pltpu.VMEM((NUM_FIELD * ROWS_W,), jnp.float32),
            pltpu.VMEM((ROWS_W,), jnp.float32),
            pltpu.VMEM((L,), jnp.float32),
            pltpu.SemaphoreType.DMA,
            pltpu.SemaphoreType.DMA,
        ],
    )
    return gfn(idx_r, fv_r, fw_r, tab, bias_r)


def kernel(feat_index, feat_value, first_weights, feat_embeddings, bias):
    idx_r = feat_index.reshape(NW, NCHUNK, IDX_PER_CHUNK)
    fv_r = (feat_value.T.reshape(NUM_FIELD, NW, ROWS_W)
            .transpose(1, 0, 2).reshape(NW, NUM_FIELD * ROWS_W))
    fw_r = first_weights.reshape(-1)
    embT = feat_embeddings.T
    tail = feat_embeddings[TAIL_START:].reshape(-1)
    bias_r = jnp.broadcast_to(bias, (L,))
    out = _run(idx_r, fv_r, fw_r, embT, tail, bias_r)
    return out[:, None]


# speedup vs baseline: 1.9528x; 1.0132x over previous
import functools

import jax
import jax.numpy as jnp
from jax import lax
from jax.experimental import pallas as pl
from jax.experimental.pallas import tpu as pltpu
from jax.experimental.pallas import tpu_sc as plsc

NUM_FEAT = 1000000
NUM_FIELD = 26
EMBED = 16
BATCH = 16384

NC = 2
NS = 16
L = 16
NW = NC * NS

ROWS_W = BATCH // NW
CHUNK = 64
NCHUNK = ROWS_W // CHUNK
GPC = CHUNK // L
IDX_PER_CHUNK = CHUNK * NUM_FIELD
NJ = IDX_PER_CHUNK // 128

TCOLS = 1024
FULL_CHUNKS = NUM_FEAT // TCOLS
KTOT = 32
NROUND = KTOT // 2
TAIL_START = FULL_CHUNKS * TCOLS
TAIL_ROWS = NUM_FEAT - TAIL_START
TOUT = TCOLS * EMBED


def _tbody(embT_hbm, tail_hbm, out_hbm, slab_v, out_v, tail_v,
           sem_in0, sem_in1, sem_out0, sem_out1):
    sem_in = (sem_in0, sem_in1)
    sem_out = (sem_out0, sem_out1)
    cid = lax.axis_index("c")
    sid = lax.axis_index("s")
    wid = sid * NC + cid
    lanes = lax.iota(jnp.int32, L)
    h_idx = lax.shift_right_logical(lanes, 3)
    e_idx = lax.bitwise_and(lanes, 7)

    def chunk_of(k):
        return jnp.minimum(wid + NW * k, FULL_CHUNKS - 1)

    def in_descs(k, buf, make):
        col0 = pl.multiple_of(chunk_of(k) * TCOLS, TCOLS)
        ds = []
        for h in range(2):
            src = embT_hbm.at[pl.ds(h * 8, 8), pl.ds(col0, TCOLS)]
            ds.append(make(src, slab_v.at[buf, h], sem_in[buf]))
        return ds

    def out_desc(k, buf, make):
        dst = out_hbm.at[pl.ds(pl.multiple_of(chunk_of(k) * TOUT, TOUT), TOUT)]
        return make(out_v.at[pl.ds(buf * TOUT, TOUT)], dst, sem_out[buf])

    def process(k, buf):
        for d in in_descs(k, buf, pltpu.make_async_copy):
            d.wait()
        p_vec = jnp.full((L,), buf, jnp.int32)
        obase = buf * TOUT

        def jbody(j16, _):
            rows = []
            for u in range(L):
                j_vec = jnp.full((L,), j16 * L + u, jnp.int32)
                rows.append(plsc.load_gather(
                    slab_v, [p_vec, h_idx, e_idx, j_vec]))
            for u in range(L):
                out_v[pl.ds(pl.multiple_of(
                    obase + j16 * (L * EMBED) + u * EMBED, L), L)] = rows[u]
            return 0

        lax.fori_loop(0, TCOLS // L, jbody, 0)
        out_desc(k, buf, pltpu.async_copy)

    in_descs(0, 0, pltpu.async_copy)
    in_descs(1, 1, pltpu.async_copy)

    def round_body(r, _):
        for b in range(2):
            k = 2 * r + b

            @pl.when(r > 0)
            def _(k=k, b=b):
                out_desc(k - 2, b, pltpu.make_async_copy).wait()

            process(k, b)
            in_descs(k + 2, b, pltpu.async_copy)
        return 0

    lax.fori_loop(0, NROUND, round_body, 0)
    for b in range(2):
        out_desc(KTOT - 2 + b, b, pltpu.make_async_copy).wait()
        for d in in_descs(KTOT + b, b, pltpu.make_async_copy):
            d.wait()

    @pl.when(wid == NW - 1)
    def _():
        pltpu.sync_copy(tail_hbm, tail_v)
        pltpu.sync_copy(tail_v,
                        out_hbm.at[pl.ds(TAIL_START * EMBED,
                                         TAIL_ROWS * EMBED)])


def _gbody(idx_hbm, fv_hbm, fw_hbm, emb_hbm, bias_hbm, out_hbm,
           idx_v, rows_v, fwr_v, fv_v, out_v, bias_v, sem0, sem1):
    sem = (sem0, sem1)
    cid = lax.axis_index("c")
    sid = lax.axis_index("s")
    wid = sid * NC + cid

    pltpu.sync_copy(fv_hbm.at[wid], fv_v)
    pltpu.sync_copy(idx_hbm.at[wid], idx_v)
    pltpu.sync_copy(bias_hbm, bias_v)
    bias_vec = bias_v[...]
    lanes = lax.iota(jnp.int32, L)
    zero = jnp.zeros((L,), jnp.float32)

    def fire_chunk(c_i, buf):
        idx_c = idx_v.at[pl.ds(c_i * IDX_PER_CHUNK, IDX_PER_CHUNK)]
        return [
            pltpu.async_copy(emb_hbm.at[idx_c], rows_v.at[buf], sem[buf]),
            pltpu.async_copy(fw_hbm.at[idx_c], fwr_v.at[buf], sem[buf]),
        ]

    in_flight = [None, None]
    in_flight[0] = fire_chunk(0, 0)

    for c_i in range(NCHUNK):
        p = c_i % 2
        if c_i + 1 < NCHUNK:
            in_flight[1 - p] = fire_chunk(c_i + 1, 1 - p)
        for cp in in_flight[p]:
            cp.wait()
        p_vec = jnp.full((L,), p, jnp.int32)

        def group_body(g, _, c_i=c_i, p_vec=p_vec):
            base_l = g * L

            def f_body(f, carry):
                y1, ssq = carry[0], carry[1]
                ss = carry[2:]
                p_pos = (base_l + lanes) * NUM_FIELD + f
                off = pl.multiple_of(f * ROWS_W + c_i * CHUNK + base_l, L)
                vf = fv_v[pl.ds(off, L)]
                fwv = plsc.load_gather(fwr_v, [p_vec, p_pos])
                rs = []
                for e in range(EMBED):
                    ee = jnp.full((L,), e, jnp.int32)
                    rs.append(plsc.load_gather(rows_v, [p_vec, p_pos, ee]))
                y1 = y1 + fwv * vf
                new_ss = []
                for e in range(EMBED):
                    t = rs[e] * vf
                    new_ss.append(ss[e] + t)
                    ssq = ssq + t * t
                return (y1, ssq) + tuple(new_ss)

            init = (zero, zero) + tuple(zero for _ in range(EMBED))
            res = lax.fori_loop(0, NUM_FIELD, f_body, init)
            y1, ssq = res[0], res[1]
            acc = zero
            for e in range(EMBED):
                acc = acc + res[2 + e] * res[2 + e]
            y2 = 0.5 * (acc - ssq)
            out_v[pl.ds(c_i * CHUNK + base_l, L)] = bias_vec + y1 + y2
            return 0

        lax.fori_loop(0, GPC, group_body, 0)

    pltpu.sync_copy(out_v, out_hbm.at[pl.ds(wid * ROWS_W, ROWS_W)])


@jax.jit
def _run(idx_r, fv_r, fw_r, embT, tail, bias_r):
    mesh = plsc.VectorSubcoreMesh(core_axis_name="c", subcore_axis_name="s")
    tfn = pl.kernel(
        _tbody,
        out_type=jax.ShapeDtypeStruct((NUM_FEAT * EMBED,), jnp.float32),
        mesh=mesh,
        compiler_params=pltpu.CompilerParams(
            needs_layout_passes=False, use_tc_tiling_on_sc=True),
        scratch_types=[
            pltpu.VMEM((2, 2, 8, TCOLS), jnp.float32),
            pltpu.VMEM((2 * TCOLS * EMBED,), jnp.float32),
            pltpu.VMEM((TAIL_ROWS * EMBED,), jnp.float32),
            pltpu.SemaphoreType.DMA,
            pltpu.SemaphoreType.DMA,
            pltpu.SemaphoreType.DMA,
            pltpu.SemaphoreType.DMA,
        ],
    )
    tab_lin = tfn(embT, tail)
    tab = tab_lin.reshape(NUM_FEAT, EMBED)

    gfn = pl.kernel(
        _gbody,
        out_type=jax.ShapeDtypeStruct((BATCH,), jnp.float32),
        mesh=mesh,
        compiler_params=pltpu.CompilerParams(
            needs_layout_passes=False, use_tc_tiling_on_sc=False),
        scratch_types=[
            pltpu.VMEM((NCHUNK * IDX_PER_CHUNK,), jnp.int32),
            pltpu.VMEM((2, IDX_PER_CHUNK, EMBED), jnp.float32),
            pltpu.VMEM((2, IDX_PER_CHUNK), jnp.float32),
            pltpu.VMEM((NUM_FIELD * ROWS_W,), jnp.float32),
            pltpu.VMEM((ROWS_W,), jnp.float32),
            pltpu.VMEM((L,), jnp.float32),
            pltpu.SemaphoreType.DMA,
            pltpu.SemaphoreType.DMA,
        ],
    )
    return gfn(idx_r, fv_r, fw_r, tab, bias_r)


def kernel(feat_index, feat_value, first_weights, feat_embeddings, bias):
    idx_r = feat_index.reshape(NW, NCHUNK * IDX_PER_CHUNK)
    fv_r = (feat_value.T.reshape(NUM_FIELD, NW, ROWS_W)
            .transpose(1, 0, 2).reshape(NW, NUM_FIELD * ROWS_W))
    fw_r = first_weights.reshape(-1)
    embT = feat_embeddings.T
    tail = feat_embeddings[TAIL_START:].reshape(-1)
    bias_r = jnp.broadcast_to(bias, (L,))
    out = _run(idx_r, fv_r, fw_r, embT, tail, bias_r)
    return out[:, None]
